# Initial kernel scaffold; baseline (speedup 1.0000x reference)
import functools

import jax
import jax.numpy as jnp
from jax import lax
from jax.experimental import pallas as pl
from jax.experimental.pallas import tpu as pltpu
from jax.experimental.pallas import tpu_sc as plsc

N = 50000
E = 800000
D = 64
G = 512
NH = 25088
ACC_R = NH + 16
NPAD = 53248
BLK = 512
EPT = 50048
EP = 16 * EPT
CH = 128
NCH = EPT // CH
ZR = 523
EPS = 1e-5


def _make_agg(df):
    mesh = plsc.VectorSubcoreMesh(core_axis_name="c", subcore_axis_name="s")

    @functools.partial(
        pl.kernel,
        out_type=jax.ShapeDtypeStruct((NPAD, df), jnp.float32),
        mesh=mesh,
        scratch_types=[
            pltpu.VMEM_SHARED((ACC_R, df), jnp.float32),
            pltpu.VMEM((CH,), jnp.int32),
            pltpu.VMEM((CH,), jnp.int32),
            pltpu.VMEM((CH,), jnp.int32),
            pltpu.VMEM((CH, df), jnp.float32),
            pltpu.VMEM((ZR, df), jnp.float32),
            pltpu.SemaphoreType.DMA,
        ],
    )
    def agg(table, src, dst, zeros, out, acc, sidx, didx, lidx, rows, zv, sem):
        c = lax.axis_index("c")
        s = lax.axis_index("s")
        pltpu.sync_copy(zeros, zv)
        for r in range(3):
            pltpu.sync_copy(zv, acc.at[pl.ds((s * 3 + r) * ZR, ZR)])
        plsc.subcore_barrier()
        off = c * NH

        def chunk(k, carry):
            base = s * EPT + k * CH
            pltpu.sync_copy(src.at[pl.ds(base, CH)], sidx)
            pltpu.sync_copy(dst.at[pl.ds(base, CH)], didx)
            for j in range(CH // 16):
                dvec = didx[pl.ds(j * 16, 16)]
                loc = dvec - off
                okv = (loc >= 0) & (loc < NH)
                lidx[pl.ds(j * 16, 16)] = jnp.where(okv, loc, NH)
            pltpu.async_copy(table.at[sidx], rows, sem).wait()
            pltpu.sync_copy(rows, acc.at[lidx], add=True)
            return carry

        lax.fori_loop(0, NCH, chunk, 0)
        plsc.subcore_barrier()
        for r in range(4):
            row0 = s * 1568 + r * 392
            pltpu.sync_copy(acc.at[pl.ds(row0, 392)], zv.at[pl.ds(0, 392)])
            pltpu.sync_copy(zv.at[pl.ds(0, 392)], out.at[pl.ds(off + row0, 392)])

    return agg


_agg16 = _make_agg(16)
_agg64 = _make_agg(64)


def _make_pool():
    mesh = plsc.VectorSubcoreMesh(core_axis_name="c", subcore_axis_name="s")

    @functools.partial(
        pl.kernel,
        out_type=[
            jax.ShapeDtypeStruct((2 * G, D), jnp.float32),
            jax.ShapeDtypeStruct((2 * G, 16), jnp.float32),
        ],
        mesh=mesh,
        scratch_types=[
            pltpu.VMEM_SHARED((G + 16, D), jnp.float32),
            pltpu.VMEM_SHARED((G + 16, 16), jnp.float32),
            pltpu.VMEM((112, D), jnp.float32),
            pltpu.VMEM((112,), jnp.int32),
            pltpu.VMEM((112, 16), jnp.float32),
            pltpu.VMEM((33, D), jnp.float32),
            pltpu.VMEM((33, 16), jnp.float32),
        ],
    )
    def pool(table, batch, ones_h, zp_h, zc_h, outp, outc,
             accp, accc, prow, bidx, ones_v, zp, zc):
        c = lax.axis_index("c")
        s = lax.axis_index("s")
        pltpu.sync_copy(ones_h, ones_v)
        pltpu.sync_copy(zp_h, zp)
        pltpu.sync_copy(zp, accp.at[pl.ds(s * 33, 33)])
        pltpu.sync_copy(zc_h, zc)
        pltpu.sync_copy(zc, accc.at[pl.ds(s * 33, 33)])
        plsc.subcore_barrier()
        w = c * 16 + s

        def chunk(k, carry):
            row0 = w * 1568 + k * 112
            pltpu.sync_copy(batch.at[pl.ds(row0, 112)], bidx)
            pltpu.sync_copy(table.at[pl.ds(row0, 112)], prow)
            pltpu.sync_copy(prow, accp.at[bidx], add=True)
            pltpu.sync_copy(ones_v, accc.at[bidx], add=True)
            return carry

        lax.fori_loop(0, 14, chunk, 0)
        plsc.subcore_barrier()
        row0 = s * 32
        pltpu.sync_copy(accp.at[pl.ds(row0, 32)], prow.at[pl.ds(0, 32)])
        pltpu.sync_copy(prow.at[pl.ds(0, 32)], outp.at[pl.ds(c * G + row0, 32)])
        pltpu.sync_copy(accc.at[pl.ds(row0, 32)], ones_v.at[pl.ds(0, 32)])
        pltpu.sync_copy(ones_v.at[pl.ds(0, 32)], outc.at[pl.ds(c * G + row0, 32)])

    return pool


_pool = _make_pool()


def _mlp1(xw, agg16, w1w, b1, w2, b2):
    def body(x_ref, a_ref, w1_ref, b1_ref, w2_ref, b2_ref,
             z_ref, s_ref, q_ref):
        i = pl.program_id(0)
        u = x_ref[...] + a_ref[...]
        h1 = jnp.maximum(u @ w1_ref[...] + b1_ref[...], 0.0)
        h2 = jnp.maximum(h1 @ w2_ref[...] + b2_ref[...], 0.0)
        z_ref[...] = h2
        rows = i * BLK + lax.broadcasted_iota(jnp.int32, (BLK, 1), 0)
        h2m = jnp.where(rows < N, h2, 0.0)

        @pl.when(i == 0)
        def _():
            s_ref[...] = jnp.zeros_like(s_ref)
            q_ref[...] = jnp.zeros_like(q_ref)

        s_ref[...] += jnp.sum(h2m, axis=0, keepdims=True)
        q_ref[...] += jnp.sum(h2m * h2m, axis=0, keepdims=True)

    return pl.pallas_call(
        body,
        grid=(NPAD // BLK,),
        in_specs=[
            pl.BlockSpec((BLK, 16), lambda i: (i, 0)),
            pl.BlockSpec((BLK, 16), lambda i: (i, 0)),
            pl.BlockSpec((16, D), lambda i: (0, 0)),
            pl.BlockSpec((1, D), lambda i: (0, 0)),
            pl.BlockSpec((D, D), lambda i: (0, 0)),
            pl.BlockSpec((1, D), lambda i: (0, 0)),
        ],
        out_specs=[
            pl.BlockSpec((BLK, D), lambda i: (i, 0)),
            pl.BlockSpec((1, D), lambda i: (0, 0)),
            pl.BlockSpec((1, D), lambda i: (0, 0)),
        ],
        out_shape=[
            jax.ShapeDtypeStruct((NPAD, D), jnp.float32),
            jax.ShapeDtypeStruct((1, D), jnp.float32),
            jax.ShapeDtypeStruct((1, D), jnp.float32),
        ],
    )(xw, agg16, w1w, b1, w2, b2)


def _mlp23(z, agg, agg16, sums_p, sumsq_p, gamma_p, beta_p, w1, b1, w2, b2):
    def body(z_ref, a_ref, d_ref, sp_ref, qp_ref, g_ref, be_ref,
             w1_ref, b1_ref, w2_ref, b2_ref, z_out, s_ref, q_ref):
        i = pl.program_id(0)
        mean = sp_ref[...] * (1.0 / N)
        var = qp_ref[...] * (1.0 / N) - mean * mean
        sc = g_ref[...] * lax.rsqrt(var + EPS)
        tt = be_ref[...] - mean * sc
        dg = d_ref[...][:, 2:3]
        u = (z_ref[...] + a_ref[...]) * sc + (1.0 + dg) * tt
        h1 = jnp.maximum(u @ w1_ref[...] + b1_ref[...], 0.0)
        h2 = jnp.maximum(h1 @ w2_ref[...] + b2_ref[...], 0.0)
        z_out[...] = h2
        rows = i * BLK + lax.broadcasted_iota(jnp.int32, (BLK, 1), 0)
        h2m = jnp.where(rows < N, h2, 0.0)

        @pl.when(i == 0)
        def _():
            s_ref[...] = jnp.zeros_like(s_ref)
            q_ref[...] = jnp.zeros_like(q_ref)

        s_ref[...] += jnp.sum(h2m, axis=0, keepdims=True)
        q_ref[...] += jnp.sum(h2m * h2m, axis=0, keepdims=True)

    return pl.pallas_call(
        body,
        grid=(NPAD // BLK,),
        in_specs=[
            pl.BlockSpec((BLK, D), lambda i: (i, 0)),
            pl.BlockSpec((BLK, D), lambda i: (i, 0)),
            pl.BlockSpec((BLK, 16), lambda i: (i, 0)),
            pl.BlockSpec((1, D), lambda i: (0, 0)),
            pl.BlockSpec((1, D), lambda i: (0, 0)),
            pl.BlockSpec((1, D), lambda i: (0, 0)),
            pl.BlockSpec((1, D), lambda i: (0, 0)),
            pl.BlockSpec((D, D), lambda i: (0, 0)),
            pl.BlockSpec((1, D), lambda i: (0, 0)),
            pl.BlockSpec((D, D), lambda i: (0, 0)),
            pl.BlockSpec((1, D), lambda i: (0, 0)),
        ],
        out_specs=[
            pl.BlockSpec((BLK, D), lambda i: (i, 0)),
            pl.BlockSpec((1, D), lambda i: (0, 0)),
            pl.BlockSpec((1, D), lambda i: (0, 0)),
        ],
        out_shape=[
            jax.ShapeDtypeStruct((NPAD, D), jnp.float32),
            jax.ShapeDtypeStruct((1, D), jnp.float32),
            jax.ShapeDtypeStruct((1, D), jnp.float32),
        ],
    )(z, agg, agg16, sums_p, sumsq_p, gamma_p, beta_p, w1, b1, w2, b2)


def _final(pooledp, cntp, sums_p, sumsq_p, gamma_p, beta_p, w1, b1, w2p, b2p):
    def body(p_ref, c_ref, sp_ref, qp_ref, g_ref, be_ref,
             w1_ref, b1_ref, w2_ref, b2_ref, o_ref):
        mean = sp_ref[...] * (1.0 / N)
        var = qp_ref[...] * (1.0 / N) - mean * mean
        sc = g_ref[...] * lax.rsqrt(var + EPS)
        tt = be_ref[...] - mean * sc
        pp = p_ref[...]
        p = pp[0:G] + pp[G:2 * G]
        cc = c_ref[...]
        cnt = cc[0:G, 0:1] + cc[G:2 * G, 0:1]
        pn = p * sc + cnt * tt
        h = jnp.maximum(pn @ w1_ref[...] + b1_ref[...], 0.0)
        o_ref[...] = h @ w2_ref[...] + b2_ref[...]

    return pl.pallas_call(
        body,
        out_shape=jax.ShapeDtypeStruct((G, 128), jnp.float32),
    )(pooledp, cntp, sums_p, sumsq_p, gamma_p, beta_p, w1, b1, w2p, b2p)


def kernel(x, edge_index, batch, params):
    f32 = jnp.float32
    src = edge_index[0]
    dst = edge_index[1]
    src_p = jnp.concatenate([src, jnp.zeros((EP - E,), jnp.int32)])
    dst_p = jnp.concatenate([dst, jnp.full((EP - E,), 2 * NH, jnp.int32)])
    xw = jnp.pad(
        jnp.concatenate([x, jnp.ones((N, 1), f32), jnp.zeros((N, 13), f32)], axis=1),
        ((0, NPAD - N), (0, 0)))
    zeros16 = jnp.zeros((ZR, 16), f32)
    zeros64 = jnp.zeros((ZR, D), f32)

    agg16 = _agg16(xw, src_p, dst_p, zeros16)

    p1 = params["conv1"]
    w1w = jnp.pad(p1["W1"], ((0, 14), (0, 0)))
    z1, s1, q1 = _mlp1(xw, agg16, w1w, p1["b1"].reshape(1, D),
                       p1["W2"], p1["b2"].reshape(1, D))

    zc, sc_, qc = z1, s1, q1
    gprev, bprev = p1["gamma"], p1["beta"]
    for p in params["convs"]:
        agg = _agg64(zc, src_p, dst_p, zeros64)
        zc, sc_, qc = _mlp23(
            zc, agg, agg16, sc_, qc,
            gprev.reshape(1, D), bprev.reshape(1, D),
            p["W1"], p["b1"].reshape(1, D), p["W2"], p["b2"].reshape(1, D))
        gprev, bprev = p["gamma"], p["beta"]

    batch_p = jnp.concatenate([batch, jnp.full((2 * NH - N,), G, jnp.int32)])
    ones112 = jnp.ones((112, 16), f32)
    z33_64 = jnp.zeros((33, D), f32)
    z33_16 = jnp.zeros((33, 16), f32)
    pooledp, cntp = _pool(zc, batch_p, ones112, z33_64, z33_16)

    w2p = jnp.pad(params["lin2_W"], ((0, 0), (0, 127)))
    b2p = jnp.broadcast_to(params["lin2_b"].reshape(1, 1), (1, 128))
    out = _final(pooledp, cntp, sc_, qc,
                 gprev.reshape(1, D), bprev.reshape(1, D),
                 params["lin1_W"], params["lin1_b"].reshape(1, D),
                 w2p, b2p)
    return out[:, 0]


# speedup vs baseline: 3.2808x; 3.2808x over previous
import functools

import jax
import jax.numpy as jnp
from jax import lax
from jax.experimental import pallas as pl
from jax.experimental.pallas import tpu as pltpu
from jax.experimental.pallas import tpu_sc as plsc

N = 50000
E = 800000
D = 64
G = 512
NH = 25088
ACC_R = NH + 16
NPAD = 53248
BLK = 512
EPT = 50048
EP = 16 * EPT
CH = 128
NCH = EPT // CH
CB = 112
EPS = 1e-5


def _make_agg(df):
    mesh = plsc.VectorSubcoreMesh(core_axis_name="c", subcore_axis_name="s", num_cores=2, num_subcores=16)

    @functools.partial(
        pl.kernel,
        out_type=jax.ShapeDtypeStruct((NPAD, df), jnp.float32),
        mesh=mesh,
        scratch_types=[
            pltpu.VMEM_SHARED((ACC_R, df), jnp.float32),
            pltpu.VMEM((CH,), jnp.int32),
            pltpu.VMEM((CH,), jnp.int32),
            pltpu.VMEM((CH,), jnp.int32),
            pltpu.VMEM((CH, df), jnp.float32),
            pltpu.VMEM((CB, df), jnp.float32),
            pltpu.SemaphoreType.DMA,
        ],
        compiler_params=pltpu.CompilerParams(use_tc_tiling_on_sc=False),
    )
    def agg(table, src, dst, zeros, out, acc, sidx, didx, lidx, rows, zv, sem):
        c = lax.axis_index("c")
        s = lax.axis_index("s")
        pltpu.sync_copy(zeros, zv)

        def zchunk(r, carry):
            pltpu.sync_copy(zv, acc.at[pl.ds(s * 1568 + r * CB, CB)])
            return carry

        lax.fori_loop(0, 14, zchunk, 0)

        @pl.when(s == 0)
        def _():
            pltpu.sync_copy(zv.at[pl.ds(0, 16)], acc.at[pl.ds(NH, 16)])

        plsc.subcore_barrier()
        off = c * NH

        def chunk(k, carry):
            base = s * EPT + k * CH
            pltpu.sync_copy(src.at[pl.ds(base, CH)], sidx)
            pltpu.sync_copy(dst.at[pl.ds(base, CH)], didx)
            for j in range(CH // 16):
                dvec = didx[pl.ds(j * 16, 16)]
                loc = dvec - off
                okv = (loc >= 0) & (loc < NH)
                lidx[pl.ds(j * 16, 16)] = jnp.where(okv, loc, NH)
            pltpu.async_copy(table.at[sidx], rows, sem).wait()
            pltpu.sync_copy(rows, acc.at[lidx], add=True)
            return carry

        lax.fori_loop(0, NCH, chunk, 0)
        plsc.subcore_barrier()
        def ochunk(r, carry):
            row0 = s * 1568 + r * CB
            pltpu.sync_copy(acc.at[pl.ds(row0, CB)], zv)
            pltpu.sync_copy(zv, out.at[pl.ds(off + row0, CB)])
            return carry

        lax.fori_loop(0, 14, ochunk, 0)

    return agg


_agg16 = _make_agg(16)
_agg64 = _make_agg(64)


def _make_pool():
    mesh = plsc.VectorSubcoreMesh(core_axis_name="c", subcore_axis_name="s", num_cores=2, num_subcores=16)

    @functools.partial(
        pl.kernel,
        out_type=[
            jax.ShapeDtypeStruct((2 * G, D), jnp.float32),
            jax.ShapeDtypeStruct((2 * G, 16), jnp.float32),
        ],
        mesh=mesh,
        scratch_types=[
            pltpu.VMEM_SHARED((G + 16, D), jnp.float32),
            pltpu.VMEM_SHARED((G + 16, 16), jnp.float32),
            pltpu.VMEM((112, D), jnp.float32),
            pltpu.VMEM((112,), jnp.int32),
            pltpu.VMEM((112, 16), jnp.float32),
            pltpu.VMEM((33, D), jnp.float32),
            pltpu.VMEM((33, 16), jnp.float32),
        ],
        compiler_params=pltpu.CompilerParams(use_tc_tiling_on_sc=False),
    )
    def pool(table, batch, ones_h, zp_h, zc_h, outp, outc,
             accp, accc, prow, bidx, ones_v, zp, zc):
        c = lax.axis_index("c")
        s = lax.axis_index("s")
        pltpu.sync_copy(ones_h, ones_v)
        pltpu.sync_copy(zp_h, zp)
        pltpu.sync_copy(zp, accp.at[pl.ds(s * 33, 33)])
        pltpu.sync_copy(zc_h, zc)
        pltpu.sync_copy(zc, accc.at[pl.ds(s * 33, 33)])
        plsc.subcore_barrier()
        w = c * 16 + s

        def chunk(k, carry):
            row0 = w * 1568 + k * 112
            pltpu.sync_copy(batch.at[pl.ds(row0, 112)], bidx)
            pltpu.sync_copy(table.at[pl.ds(row0, 112)], prow)
            pltpu.sync_copy(prow, accp.at[bidx], add=True)
            pltpu.sync_copy(ones_v, accc.at[bidx], add=True)
            return carry

        lax.fori_loop(0, 14, chunk, 0)
        plsc.subcore_barrier()
        row0 = s * 32
        pltpu.sync_copy(accp.at[pl.ds(row0, 32)], prow.at[pl.ds(0, 32)])
        pltpu.sync_copy(prow.at[pl.ds(0, 32)], outp.at[pl.ds(c * G + row0, 32)])
        pltpu.sync_copy(accc.at[pl.ds(row0, 32)], ones_v.at[pl.ds(0, 32)])
        pltpu.sync_copy(ones_v.at[pl.ds(0, 32)], outc.at[pl.ds(c * G + row0, 32)])

    return pool


_pool = _make_pool()


def _mlp1(xw, agg16, w1w, b1, w2, b2):
    def body(x_ref, a_ref, w1_ref, b1_ref, w2_ref, b2_ref,
             z_ref, s_ref, q_ref):
        i = pl.program_id(0)
        u = x_ref[...] + a_ref[...]
        h1 = jnp.maximum(u @ w1_ref[...] + b1_ref[...], 0.0)
        h2 = jnp.maximum(h1 @ w2_ref[...] + b2_ref[...], 0.0)
        z_ref[...] = h2
        rows = i * BLK + lax.broadcasted_iota(jnp.int32, (BLK, 1), 0)
        h2m = jnp.where(rows < N, h2, 0.0)

        @pl.when(i == 0)
        def _():
            s_ref[...] = jnp.zeros_like(s_ref)
            q_ref[...] = jnp.zeros_like(q_ref)

        s_ref[...] += jnp.sum(h2m, axis=0, keepdims=True)
        q_ref[...] += jnp.sum(h2m * h2m, axis=0, keepdims=True)

    return pl.pallas_call(
        body,
        grid=(NPAD // BLK,),
        in_specs=[
            pl.BlockSpec((BLK, 16), lambda i: (i, 0)),
            pl.BlockSpec((BLK, 16), lambda i: (i, 0)),
            pl.BlockSpec((16, D), lambda i: (0, 0)),
            pl.BlockSpec((1, D), lambda i: (0, 0)),
            pl.BlockSpec((D, D), lambda i: (0, 0)),
            pl.BlockSpec((1, D), lambda i: (0, 0)),
        ],
        out_specs=[
            pl.BlockSpec((BLK, D), lambda i: (i, 0)),
            pl.BlockSpec((1, D), lambda i: (0, 0)),
            pl.BlockSpec((1, D), lambda i: (0, 0)),
        ],
        out_shape=[
            jax.ShapeDtypeStruct((NPAD, D), jnp.float32),
            jax.ShapeDtypeStruct((1, D), jnp.float32),
            jax.ShapeDtypeStruct((1, D), jnp.float32),
        ],
    )(xw, agg16, w1w, b1, w2, b2)


def _mlp23(z, agg, agg16, sums_p, sumsq_p, gamma_p, beta_p, w1, b1, w2, b2):
    def body(z_ref, a_ref, d_ref, sp_ref, qp_ref, g_ref, be_ref,
             w1_ref, b1_ref, w2_ref, b2_ref, z_out, s_ref, q_ref):
        i = pl.program_id(0)
        mean = sp_ref[...] * (1.0 / N)
        var = qp_ref[...] * (1.0 / N) - mean * mean
        sc = g_ref[...] * lax.rsqrt(var + EPS)
        tt = be_ref[...] - mean * sc
        dg = d_ref[...][:, 2:3]
        u = (z_ref[...] + a_ref[...]) * sc + (1.0 + dg) * tt
        h1 = jnp.maximum(u @ w1_ref[...] + b1_ref[...], 0.0)
        h2 = jnp.maximum(h1 @ w2_ref[...] + b2_ref[...], 0.0)
        z_out[...] = h2
        rows = i * BLK + lax.broadcasted_iota(jnp.int32, (BLK, 1), 0)
        h2m = jnp.where(rows < N, h2, 0.0)

        @pl.when(i == 0)
        def _():
            s_ref[...] = jnp.zeros_like(s_ref)
            q_ref[...] = jnp.zeros_like(q_ref)

        s_ref[...] += jnp.sum(h2m, axis=0, keepdims=True)
        q_ref[...] += jnp.sum(h2m * h2m, axis=0, keepdims=True)

    return pl.pallas_call(
        body,
        grid=(NPAD // BLK,),
        in_specs=[
            pl.BlockSpec((BLK, D), lambda i: (i, 0)),
            pl.BlockSpec((BLK, D), lambda i: (i, 0)),
            pl.BlockSpec((BLK, 16), lambda i: (i, 0)),
            pl.BlockSpec((1, D), lambda i: (0, 0)),
            pl.BlockSpec((1, D), lambda i: (0, 0)),
            pl.BlockSpec((1, D), lambda i: (0, 0)),
            pl.BlockSpec((1, D), lambda i: (0, 0)),
            pl.BlockSpec((D, D), lambda i: (0, 0)),
            pl.BlockSpec((1, D), lambda i: (0, 0)),
            pl.BlockSpec((D, D), lambda i: (0, 0)),
            pl.BlockSpec((1, D), lambda i: (0, 0)),
        ],
        out_specs=[
            pl.BlockSpec((BLK, D), lambda i: (i, 0)),
            pl.BlockSpec((1, D), lambda i: (0, 0)),
            pl.BlockSpec((1, D), lambda i: (0, 0)),
        ],
        out_shape=[
            jax.ShapeDtypeStruct((NPAD, D), jnp.float32),
            jax.ShapeDtypeStruct((1, D), jnp.float32),
            jax.ShapeDtypeStruct((1, D), jnp.float32),
        ],
    )(z, agg, agg16, sums_p, sumsq_p, gamma_p, beta_p, w1, b1, w2, b2)


def _final(pooledp, cntp, sums_p, sumsq_p, gamma_p, beta_p, w1, b1, w2p, b2p):
    def body(p_ref, c_ref, sp_ref, qp_ref, g_ref, be_ref,
             w1_ref, b1_ref, w2_ref, b2_ref, o_ref):
        mean = sp_ref[...] * (1.0 / N)
        var = qp_ref[...] * (1.0 / N) - mean * mean
        sc = g_ref[...] * lax.rsqrt(var + EPS)
        tt = be_ref[...] - mean * sc
        pp = p_ref[...]
        p = pp[0:G] + pp[G:2 * G]
        cc = c_ref[...]
        cnt = cc[0:G, 0:1] + cc[G:2 * G, 0:1]
        pn = p * sc + cnt * tt
        h = jnp.maximum(pn @ w1_ref[...] + b1_ref[...], 0.0)
        o_ref[...] = h @ w2_ref[...] + b2_ref[...]

    return pl.pallas_call(
        body,
        out_shape=jax.ShapeDtypeStruct((G, 128), jnp.float32),
    )(pooledp, cntp, sums_p, sumsq_p, gamma_p, beta_p, w1, b1, w2p, b2p)


def kernel(x, edge_index, batch, params):
    f32 = jnp.float32
    src = edge_index[0]
    dst = edge_index[1]
    src_p = jnp.concatenate([src, jnp.zeros((EP - E,), jnp.int32)])
    dst_p = jnp.concatenate([dst, jnp.full((EP - E,), 2 * NH, jnp.int32)])
    xw = jnp.pad(
        jnp.concatenate([x, jnp.ones((N, 1), f32), jnp.zeros((N, 13), f32)], axis=1),
        ((0, NPAD - N), (0, 0)))
    zeros16 = jnp.zeros((CB, 16), f32)
    zeros64 = jnp.zeros((CB, D), f32)

    agg16 = _agg16(xw, src_p, dst_p, zeros16)

    p1 = params["conv1"]
    w1w = jnp.pad(p1["W1"], ((0, 14), (0, 0)))
    z1, s1, q1 = _mlp1(xw, agg16, w1w, p1["b1"].reshape(1, D),
                       p1["W2"], p1["b2"].reshape(1, D))

    zc, sc_, qc = z1, s1, q1
    gprev, bprev = p1["gamma"], p1["beta"]
    for p in params["convs"]:
        agg = _agg64(zc, src_p, dst_p, zeros64)
        zc, sc_, qc = _mlp23(
            zc, agg, agg16, sc_, qc,
            gprev.reshape(1, D), bprev.reshape(1, D),
            p["W1"], p["b1"].reshape(1, D), p["W2"], p["b2"].reshape(1, D))
        gprev, bprev = p["gamma"], p["beta"]

    batch_p = jnp.concatenate([batch, jnp.full((2 * NH - N,), G, jnp.int32)])
    ones112 = jnp.ones((112, 16), f32)
    z33_64 = jnp.zeros((33, D), f32)
    z33_16 = jnp.zeros((33, 16), f32)
    pooledp, cntp = _pool(zc, batch_p, ones112, z33_64, z33_16)

    w2p = jnp.pad(params["lin2_W"], ((0, 0), (0, 127)))
    b2p = jnp.broadcast_to(params["lin2_b"].reshape(1, 1), (1, 128))
    out = _final(pooledp, cntp, sc_, qc,
                 gprev.reshape(1, D), bprev.reshape(1, D),
                 params["lin1_W"], params["lin1_b"].reshape(1, D),
                 w2p, b2p)
    return out[:, 0]
